# Initial kernel scaffold; baseline (speedup 1.0000x reference)
import jax
import jax.numpy as jnp
from jax.experimental import pallas as pl


def kernel(query_xyz, support_xyz, features):
    raise NotImplementedError("write your pallas kernel here")


# speedup vs baseline: 5.7164x; 5.7164x over previous
import jax
import jax.numpy as jnp
from jax import lax
from jax.experimental import pallas as pl
from jax.experimental.pallas import tpu as pltpu
from jax.experimental.pallas import tpu_sc as plsc

RADIUS2 = 0.1 * 0.1
K = 32
B = 4
NQ = 1024
N = 8192
C = 128
NWORK = 32
QPW = (B * NQ) // NWORK
L = 16


def _sc_body(qt, st, ff, out_xyz, out_rows,
             sxv, syv, szv, qxv, qyv, qzv,
             idxf, dxa, dya, dza, cref, rows, sem):
    cid = lax.axis_index("c")
    sid = lax.axis_index("s")
    wid = sid * 2 + cid
    b = wid // (NQ // QPW)
    q0 = (wid % (NQ // QPW)) * QPW
    gbase = b * N
    r0 = wid * (QPW * K)

    pltpu.sync_copy(st.at[pl.ds(b * 3 * N, N)], sxv)
    pltpu.sync_copy(st.at[pl.ds(b * 3 * N + N, N)], syv)
    pltpu.sync_copy(st.at[pl.ds(b * 3 * N + 2 * N, N)], szv)
    pltpu.sync_copy(qt.at[pl.ds(b * 3 * NQ + q0, QPW)], qxv)
    pltpu.sync_copy(qt.at[pl.ds(b * 3 * NQ + NQ + q0, QPW)], qyv)
    pltpu.sync_copy(qt.at[pl.ds(b * 3 * NQ + 2 * NQ + q0, QPW)], qzv)

    lanes = lax.iota(jnp.int32, L)

    def group_body(g, _):
        qx = qxv[pl.ds(g * L, L)]
        qy = qyv[pl.ds(g * L, L)]
        qz = qzv[pl.ds(g * L, L)]
        qrow = g * L + lanes
        qflat = qrow * K
        cref[...] = jnp.zeros((L,), jnp.int32)

        def pt_body(n, _c):
            nv = jnp.full((L,), n, jnp.int32)
            sx = plsc.load_gather(sxv, [nv])
            sy = plsc.load_gather(syv, [nv])
            sz = plsc.load_gather(szv, [nv])
            dx = sx - qx
            dy = sy - qy
            dz = sz - qz
            d2 = dx * dx + dy * dy + dz * dz
            near = d2 < RADIUS2

            @pl.when(jnp.any(near))
            def _():
                counts = cref[...]
                hit = near & (counts < K)
                slot = qflat + counts
                plsc.store_scatter(idxf, [slot],
                                   jnp.full((L,), gbase + n, jnp.int32),
                                   mask=hit)
                plsc.store_scatter(dxa, [slot], dx, mask=hit)
                plsc.store_scatter(dya, [slot], dy, mask=hit)
                plsc.store_scatter(dza, [slot], dz, mask=hit)
                cref[...] = counts + jnp.where(hit, 1, 0).astype(jnp.int32)

            return 0

        lax.fori_loop(0, N, pt_body, 0, unroll=4)
        counts = cref[...]

        has = counts > 0
        zeros = jnp.zeros((L,), jnp.int32)
        first_i = jnp.where(has, plsc.load_gather(idxf, [qflat]),
                            jnp.full((L,), gbase, jnp.int32))
        s0x = plsc.load_gather(sxv, [zeros])
        s0y = plsc.load_gather(syv, [zeros])
        s0z = plsc.load_gather(szv, [zeros])
        fdx = jnp.where(has, plsc.load_gather(dxa, [qflat]), s0x - qx)
        fdy = jnp.where(has, plsc.load_gather(dya, [qflat]), s0y - qy)
        fdz = jnp.where(has, plsc.load_gather(dza, [qflat]), s0z - qz)

        @pl.when(jnp.any(counts < K))
        def _():
            def fill_body(j, _):
                fill = counts <= j
                slot = qflat + j
                plsc.store_scatter(idxf, [slot], first_i, mask=fill)
                plsc.store_scatter(dxa, [slot], fdx, mask=fill)
                plsc.store_scatter(dya, [slot], fdy, mask=fill)
                plsc.store_scatter(dza, [slot], fdz, mask=fill)
                return 0

            lax.fori_loop(0, K, fill_body, 0)

        return 0

    lax.fori_loop(0, QPW // L, group_body, 0)

    pltpu.sync_copy(dxa, out_xyz.at[pl.ds((b * 3 + 0) * NQ * K + q0 * K, QPW * K)])
    pltpu.sync_copy(dya, out_xyz.at[pl.ds((b * 3 + 1) * NQ * K + q0 * K, QPW * K)])
    pltpu.sync_copy(dza, out_xyz.at[pl.ds((b * 3 + 2) * NQ * K + q0 * K, QPW * K)])

    def gather_body(ci, _):
        idxc = idxf.at[pl.ds(ci * 128, 128)]
        pltpu.async_copy(ff.at[idxc], rows, sem).wait()
        pltpu.sync_copy(rows, out_rows.at[pl.ds(r0 + ci * 128, 128)])
        return 0

    lax.fori_loop(0, (QPW * K) // 128, gather_body, 0)


def _sc_call(qt, st, ff):
    mesh = plsc.VectorSubcoreMesh(core_axis_name="c", subcore_axis_name="s",
                                  num_cores=2, num_subcores=16)
    return pl.kernel(
        _sc_body,
        out_type=(
            jax.ShapeDtypeStruct((B * 3 * NQ * K,), jnp.float32),
            jax.ShapeDtypeStruct((B * NQ * K, C), jnp.float32),
        ),
        mesh=mesh,
        compiler_params=pltpu.CompilerParams(needs_layout_passes=False),
        scratch_types=[
            pltpu.VMEM((N,), jnp.float32),
            pltpu.VMEM((N,), jnp.float32),
            pltpu.VMEM((N,), jnp.float32),
            pltpu.VMEM((QPW,), jnp.float32),
            pltpu.VMEM((QPW,), jnp.float32),
            pltpu.VMEM((QPW,), jnp.float32),
            pltpu.VMEM((QPW * K,), jnp.int32),
            pltpu.VMEM((QPW * K,), jnp.float32),
            pltpu.VMEM((QPW * K,), jnp.float32),
            pltpu.VMEM((QPW * K,), jnp.float32),
            pltpu.VMEM((L,), jnp.int32),
            pltpu.VMEM((128, C), jnp.float32),
            pltpu.SemaphoreType.DMA,
        ],
    )(qt, st, ff)


def kernel(query_xyz, support_xyz, features):
    qt = jnp.transpose(query_xyz, (0, 2, 1)).reshape(-1)
    st = jnp.transpose(support_xyz, (0, 2, 1)).reshape(-1)
    ff = jnp.transpose(features, (0, 2, 1)).reshape(B * N, C)
    out_xyz, out_rows = _sc_call(qt, st, ff)
    grouped_xyz = out_xyz.reshape(B, 3, NQ, K)
    grouped_features = jnp.transpose(
        out_rows.reshape(B, NQ, K, C), (0, 3, 1, 2))
    return (grouped_xyz, grouped_features)


# speedup vs baseline: 33.2484x; 5.8163x over previous
import jax
import jax.numpy as jnp
from jax import lax
from jax.experimental import pallas as pl
from jax.experimental.pallas import tpu as pltpu
from jax.experimental.pallas import tpu_sc as plsc

RADIUS2 = 0.1 * 0.1
K = 32
B = 4
NQ = 1024
N = 8192
C = 128
NWORK = 32
QPW = (B * NQ) // NWORK
L = 16


def _sc_body(qt, st, ff, out_xyz, out_rows,
             sxv, syv, szv, qxv, qyv, qzv,
             idxf, idxg, dxa, dya, dza, rows0, rows1, sem0, sem1):
    cid = lax.axis_index("c")
    sid = lax.axis_index("s")
    wid = sid * 2 + cid
    b = wid // (NQ // QPW)
    q0 = (wid % (NQ // QPW)) * QPW
    gbase = b * N
    r0 = wid * (QPW * K)

    pltpu.sync_copy(st.at[pl.ds(b * 3 * N, N)], sxv)
    pltpu.sync_copy(st.at[pl.ds(b * 3 * N + N, N)], syv)
    pltpu.sync_copy(st.at[pl.ds(b * 3 * N + 2 * N, N)], szv)
    pltpu.sync_copy(qt.at[pl.ds(b * 3 * NQ + q0, QPW)], qxv)
    pltpu.sync_copy(qt.at[pl.ds(b * 3 * NQ + NQ + q0, QPW)], qyv)
    pltpu.sync_copy(qt.at[pl.ds(b * 3 * NQ + 2 * NQ + q0, QPW)], qzv)

    lanes = lax.iota(jnp.int32, L)
    zeros = jnp.zeros((L,), jnp.int32)
    ones = jnp.full((L,), 1, jnp.int32)

    NG = 8

    def group_pass_body(gp, _):
        qs = []
        for t in range(NG):
            g = gp * NG + t
            qs.append((qxv[pl.ds(g * L, L)],
                       qyv[pl.ds(g * L, L)],
                       qzv[pl.ds(g * L, L)],
                       (g * L + lanes) * K))

        def pt_body(n, carry):
            nv = carry[0]
            cs = list(carry[1:])
            sx = plsc.load_gather(sxv, [nv])
            sy = plsc.load_gather(syv, [nv])
            sz = plsc.load_gather(szv, [nv])
            for t in range(NG):
                qx, qy, qz, qflat = qs[t]
                dx = sx - qx
                dy = sy - qy
                dz = sz - qz
                d2 = dx * dx + dy * dy + dz * dz
                hit = (d2 < RADIUS2) & (cs[t] < K)
                plsc.store_scatter(idxf, [qflat + cs[t]], nv, mask=hit)
                cs[t] = cs[t] + jnp.where(hit, ones, zeros)
            return tuple([nv + 1] + cs)

        out = lax.fori_loop(0, N, pt_body, tuple([zeros] * (NG + 1)),
                            unroll=4)

        for t in range(NG):
            qflat = qs[t][3]
            counts = out[1 + t]
            first_i = jnp.where(counts > 0, plsc.load_gather(idxf, [qflat]),
                                zeros)

            def fill_body(j, _, qflat=qflat, counts=counts, first_i=first_i):
                plsc.store_scatter(idxf, [qflat + j], first_i,
                                   mask=counts <= j)
                return 0

            lax.fori_loop(0, K, fill_body, 0, unroll=8)
        return 0

    lax.fori_loop(0, QPW // L // NG, group_pass_body, 0)

    gvec = jnp.full((L,), gbase, jnp.int32)

    def post_body(q, _):
        qv = jnp.full((L,), q, jnp.int32)
        qxs = plsc.load_gather(qxv, [qv])
        qys = plsc.load_gather(qyv, [qv])
        qzs = plsc.load_gather(qzv, [qv])
        for h in range(K // L):
            base = q * K + h * L
            il = idxf[pl.ds(base, L)]
            sx = plsc.load_gather(sxv, [il])
            sy = plsc.load_gather(syv, [il])
            sz = plsc.load_gather(szv, [il])
            dxa[pl.ds(base, L)] = sx - qxs
            dya[pl.ds(base, L)] = sy - qys
            dza[pl.ds(base, L)] = sz - qzs
            idxg[pl.ds(base, L)] = il + gvec
        return 0

    lax.fori_loop(0, QPW, post_body, 0)

    pltpu.sync_copy(dxa, out_xyz.at[pl.ds((b * 3 + 0) * NQ * K + q0 * K, QPW * K)])
    pltpu.sync_copy(dya, out_xyz.at[pl.ds((b * 3 + 1) * NQ * K + q0 * K, QPW * K)])
    pltpu.sync_copy(dza, out_xyz.at[pl.ds((b * 3 + 2) * NQ * K + q0 * K, QPW * K)])

    nchunk = (QPW * K) // 128

    def gdesc(ci, buf, sem):
        return pltpu.make_async_copy(ff.at[idxg.at[pl.ds(ci * 128, 128)]],
                                     buf, sem)

    gdesc(0, rows0, sem0).start()
    gdesc(1, rows1, sem1).start()

    def pair_body(cp, _):
        ci0 = cp * 2
        ci1 = cp * 2 + 1
        for ci, buf, sem in ((ci0, rows0, sem0), (ci1, rows1, sem1)):
            gdesc(ci, buf, sem).wait()
            pltpu.sync_copy(buf, out_rows.at[pl.ds(r0 + ci * 128, 128)])

            @pl.when(ci + 2 < nchunk)
            def _(ci=ci, buf=buf, sem=sem):
                gdesc(ci + 2, buf, sem).start()

        return 0

    lax.fori_loop(0, nchunk // 2, pair_body, 0)


def _sc_call(qt, st, ff):
    mesh = plsc.VectorSubcoreMesh(core_axis_name="c", subcore_axis_name="s",
                                  num_cores=2, num_subcores=16)
    return pl.kernel(
        _sc_body,
        out_type=(
            jax.ShapeDtypeStruct((B * 3 * NQ * K,), jnp.float32),
            jax.ShapeDtypeStruct((B * NQ * K, C), jnp.float32),
        ),
        mesh=mesh,
        compiler_params=pltpu.CompilerParams(needs_layout_passes=False),
        scratch_types=[
            pltpu.VMEM((N,), jnp.float32),
            pltpu.VMEM((N,), jnp.float32),
            pltpu.VMEM((N,), jnp.float32),
            pltpu.VMEM((QPW,), jnp.float32),
            pltpu.VMEM((QPW,), jnp.float32),
            pltpu.VMEM((QPW,), jnp.float32),
            pltpu.VMEM((QPW * K,), jnp.int32),
            pltpu.VMEM((QPW * K,), jnp.int32),
            pltpu.VMEM((QPW * K,), jnp.float32),
            pltpu.VMEM((QPW * K,), jnp.float32),
            pltpu.VMEM((QPW * K,), jnp.float32),
            pltpu.VMEM((128, C), jnp.float32),
            pltpu.VMEM((128, C), jnp.float32),
            pltpu.SemaphoreType.DMA,
            pltpu.SemaphoreType.DMA,
        ],
    )(qt, st, ff)


def kernel(query_xyz, support_xyz, features):
    qt = jnp.transpose(query_xyz, (0, 2, 1)).reshape(-1)
    st = jnp.transpose(support_xyz, (0, 2, 1)).reshape(-1)
    ff = jnp.transpose(features, (0, 2, 1)).reshape(B * N, C)
    out_xyz, out_rows = _sc_call(qt, st, ff)
    grouped_xyz = out_xyz.reshape(B, 3, NQ, K)
    grouped_features = jnp.transpose(
        out_rows.reshape(B, NQ, K, C), (0, 3, 1, 2))
    return (grouped_xyz, grouped_features)
